# baseline (speedup 1.0000x reference)
import functools

import jax
import jax.numpy as jnp
from jax import lax
from jax.experimental import pallas as pl
from jax.experimental.pallas import tpu as pltpu
from jax.experimental.pallas import tpu_sc as plsc

VOCAB = 1000000
D = 64
B = 4096 * 200
PAD = 0

NC, NS, L = 2, 16, 16
NW = NC * NS
BPW = B // NW
CHUNK = 512
NCHUNK = BPW // CHUNK
SUB = 128
NSUB = CHUNK // SUB
GPC = CHUNK // L


@functools.partial(
    pl.kernel,
    out_type=jax.ShapeDtypeStruct((B, D), jnp.float32),
    mesh=plsc.VectorSubcoreMesh(core_axis_name="c", subcore_axis_name="s"),
    scratch_types=[
        pltpu.VMEM((CHUNK,), jnp.int32),
        pltpu.VMEM((CHUNK, D), jnp.float32),
        pltpu.SemaphoreType.DMA,
    ],
    compiler_params=pltpu.CompilerParams(
        needs_layout_passes=False, use_tc_tiling_on_sc=False
    ),
)
def _emb_lookup(idx_hbm, table_hbm, out_hbm, idx_v, rows_v, sem):
    wid = lax.axis_index("s") * NC + lax.axis_index("c")
    lane = lax.iota(jnp.int32, L)
    zeros = jnp.zeros((L,), jnp.float32)

    def chunk_body(i, carry):
        base = wid * BPW + i * CHUNK
        pltpu.sync_copy(idx_hbm.at[pl.ds(base, CHUNK)], idx_v)
        copies = [
            pltpu.async_copy(
                table_hbm.at[idx_v.at[pl.ds(j * SUB, SUB)]],
                rows_v.at[pl.ds(j * SUB, SUB)],
                sem,
            )
            for j in range(NSUB)
        ]
        for c in copies:
            c.wait()

        def count(g, acc):
            m = idx_v[pl.ds(g * L, L)] == PAD
            return acc + plsc.all_reduce_population_count(m)

        total = lax.fori_loop(0, GPC, count, jnp.zeros((L,), jnp.int32))

        @pl.when(total[0] > 0)
        def _fixup():
            def grp(g, c2):
                m = idx_v[pl.ds(g * L, L)] == PAD
                rowi = g * L + lane
                for k in range(D):
                    plsc.store_scatter(
                        rows_v,
                        [rowi, jnp.full((L,), k, jnp.int32)],
                        zeros,
                        mask=m,
                    )
                return c2

            lax.fori_loop(0, GPC, grp, 0)

        pltpu.sync_copy(rows_v, out_hbm.at[pl.ds(base, CHUNK)])
        return carry

    lax.fori_loop(0, NCHUNK, chunk_body, 0)


def kernel(inputs, embedding_matrix):
    bsz, seq = inputs.shape
    idx = inputs.reshape(-1).astype(jnp.int32)
    out = _emb_lookup(idx, embedding_matrix)
    return out.reshape(bsz, seq, D)
